# Initial kernel scaffold; baseline (speedup 1.0000x reference)
import jax
import jax.numpy as jnp
from jax.experimental import pallas as pl


def kernel(element_indicies, table_element, table_meta, meta_indicies, meta_weights):
    raise NotImplementedError("write your pallas kernel here")


# speedup vs baseline: 1.8193x; 1.8193x over previous
import functools

import jax
import jax.numpy as jnp
from jax import lax
from jax.experimental import pallas as pl
from jax.experimental.pallas import tpu as pltpu
from jax.experimental.pallas import tpu_sc as plsc

B = 16384
D = 32
T = 4
NC = 2
NS = 16
NW = NC * NS
BPW = B // NW
CHUNK = 128
NIDX = BPW // CHUNK
NMETA = BPW * T
NMCH = NMETA // CHUNK

_mesh = plsc.VectorSubcoreMesh(core_axis_name="c", subcore_axis_name="s")


@functools.partial(
    pl.kernel,
    mesh=_mesh,
    out_type=jax.ShapeDtypeStruct((B, D), jnp.float32),
    compiler_params=pltpu.CompilerParams(use_tc_tiling_on_sc=False),
    scratch_types=[
        pltpu.VMEM((NIDX, CHUNK), jnp.int32),
        pltpu.VMEM((BPW, D), jnp.float32),
        pltpu.VMEM((NMCH, CHUNK), jnp.int32),
        pltpu.VMEM((NMCH, CHUNK), jnp.int32),
        pltpu.VMEM((NMCH, CHUNK), jnp.float32),
        pltpu.VMEM((NMETA, D), jnp.float32),
        pltpu.SemaphoreType.DMA,
        pltpu.SemaphoreType.DMA,
        pltpu.SemaphoreType.DMA,
        pltpu.SemaphoreType.DMA,
    ],
)
def _sc_fused_lookup(e2_hbm, te_hbm, tm_hbm, mif_hbm, mwf_hbm, out_hbm,
                     idx_v, elem_v, fidx_v, mi_v, mw_v, meta_v,
                     sem_e, sem_i, sem_w, sem_m):
    wid = lax.axis_index("s") * NC + lax.axis_index("c")
    base = wid * BPW

    pltpu.sync_copy(e2_hbm.at[pl.ds(wid * NIDX, NIDX)], idx_v)

    el_d = []
    for j in range(NIDX):
        el_d.append(pltpu.async_copy(
            te_hbm.at[idx_v.at[j]], elem_v.at[pl.ds(j * CHUNK, CHUNK)], sem_e))

    def flat_body(k, _):
        v4 = idx_v[k // 8, pl.ds((k % 8) * 16, 16)] * 4
        for t in range(T):
            fidx_v[t * NIDX + k // 8, pl.ds((k % 8) * 16, 16)] = v4 + t
        return 0

    lax.fori_loop(0, BPW // 16, flat_body, 0)

    mi_d, mw_d = [], []
    for j in range(NMCH):
        mi_d.append(pltpu.async_copy(
            mif_hbm.at[fidx_v.at[j]], mi_v.at[j], sem_i))
    for j in range(NMCH):
        mw_d.append(pltpu.async_copy(
            mwf_hbm.at[fidx_v.at[j]], mw_v.at[j], sem_w))
    for d in mi_d:
        d.wait()

    mt_d = []
    for j in range(NMCH):
        mt_d.append(pltpu.async_copy(
            tm_hbm.at[mi_v.at[j]], meta_v.at[pl.ds(j * CHUNK, CHUNK)], sem_m))
    for d in el_d:
        d.wait()
    for d in mw_d:
        d.wait()
    for d in mt_d:
        d.wait()

    scale = jnp.float32(0.2)

    def group_body(g, _):
        r = g // 8
        cb = (g % 8) * 16
        wv = [mw_v[t * NIDX + r, pl.ds(cb, 16)] for t in range(T)]
        for l in range(16):
            b = g * 16 + l
            w = [jnp.full((16,), wv[t][l]) for t in range(T)]
            for h in range(D // 16):
                sl = pl.ds(h * 16, 16)
                acc = elem_v[b, sl]
                for t in range(T):
                    acc = acc + w[t] * meta_v[t * BPW + b, sl]
                elem_v[b, sl] = acc * scale
        return 0

    lax.fori_loop(0, BPW // 16, group_body, 0)

    pltpu.sync_copy(elem_v, out_hbm.at[pl.ds(base, BPW)])


def kernel(element_indicies, table_element, table_meta, meta_indicies,
           meta_weights):
    e2 = element_indicies.reshape(B // CHUNK, CHUNK)
    return _sc_fused_lookup(e2, table_element, table_meta,
                            meta_indicies.reshape(-1),
                            meta_weights.reshape(-1))


# speedup vs baseline: 4.1149x; 2.2618x over previous
import functools

import jax
import jax.numpy as jnp
from jax import lax
from jax.experimental import pallas as pl
from jax.experimental.pallas import tpu as pltpu
from jax.experimental.pallas import tpu_sc as plsc

B = 16384
D = 32
T = 4
NE = 100000
NC = 2
NS = 16
NW = NC * NS
BPW = B // NW
CHUNK = 128
NIDX = BPW // CHUNK
NMETA = BPW * T
NMCH = NMETA // CHUNK

_mesh = plsc.VectorSubcoreMesh(core_axis_name="c", subcore_axis_name="s")


@functools.partial(
    pl.kernel,
    mesh=_mesh,
    out_type=jax.ShapeDtypeStruct((B, D), jnp.float32),
    compiler_params=pltpu.CompilerParams(use_tc_tiling_on_sc=False),
    scratch_types=[
        pltpu.VMEM((NIDX, CHUNK), jnp.int32),
        pltpu.VMEM((BPW, D), jnp.float32),
        pltpu.VMEM((NMCH, CHUNK), jnp.int32),
        pltpu.VMEM((NMCH, CHUNK), jnp.int32),
        pltpu.VMEM((NMCH, CHUNK), jnp.float32),
        pltpu.VMEM((NMETA, D), jnp.float32),
        pltpu.SemaphoreType.DMA,
        pltpu.SemaphoreType.DMA,
        pltpu.SemaphoreType.DMA,
        pltpu.SemaphoreType.DMA,
    ],
)
def _sc_fused_lookup(e2_hbm, te_hbm, tm_hbm, mif_hbm, mwf_hbm, out_hbm,
                     idx_v, elem_v, fidx_v, mi_v, mw_v, meta_v,
                     sem_e, sem_i, sem_w, sem_m):
    wid = lax.axis_index("s") * NC + lax.axis_index("c")
    base = wid * BPW

    pltpu.sync_copy(e2_hbm.at[pl.ds(wid * NIDX, NIDX)], idx_v)

    el_d = []
    for j in range(NIDX):
        el_d.append(pltpu.async_copy(
            te_hbm.at[idx_v.at[j]], elem_v.at[pl.ds(j * CHUNK, CHUNK)], sem_e))

    def flat_body(k, _):
        v = idx_v[k // 8, pl.ds((k % 8) * 16, 16)]
        for t in range(T):
            fidx_v[t * NIDX + k // 8, pl.ds((k % 8) * 16, 16)] = v + t * NE
        return 0

    lax.fori_loop(0, BPW // 16, flat_body, 0)

    mi_d, mw_d = [], []
    for j in range(NMCH):
        mi_d.append(pltpu.async_copy(
            mif_hbm.at[fidx_v.at[j]], mi_v.at[j], sem_i))
    for j in range(NMCH):
        mw_d.append(pltpu.async_copy(
            mwf_hbm.at[fidx_v.at[j]], mw_v.at[j], sem_w))
    for d in mi_d:
        d.wait()

    mt_d = []
    for j in range(NMCH):
        mt_d.append(pltpu.async_copy(
            tm_hbm.at[mi_v.at[j]], meta_v.at[pl.ds(j * CHUNK, CHUNK)], sem_m))
    for d in el_d:
        d.wait()
    for d in mw_d:
        d.wait()
    for d in mt_d:
        d.wait()

    scale = jnp.float32(0.2)

    def group_body(g, _):
        r = g // 8
        cb = (g % 8) * 16
        wv = [mw_v[t * NIDX + r, pl.ds(cb, 16)] for t in range(T)]
        for l in range(16):
            b = g * 16 + l
            w = [jnp.full((16,), wv[t][l]) for t in range(T)]
            for h in range(D // 16):
                sl = pl.ds(h * 16, 16)
                acc = elem_v[b, sl]
                for t in range(T):
                    acc = acc + w[t] * meta_v[t * BPW + b, sl]
                elem_v[b, sl] = acc * scale
        return 0

    lax.fori_loop(0, BPW // 16, group_body, 0)

    pltpu.sync_copy(elem_v, out_hbm.at[pl.ds(base, BPW)])


def kernel(element_indicies, table_element, table_meta, meta_indicies,
           meta_weights):
    e2 = element_indicies.reshape(B // CHUNK, CHUNK)
    return _sc_fused_lookup(e2, table_element, table_meta,
                            meta_indicies.T.reshape(-1),
                            meta_weights.T.reshape(-1))


# speedup vs baseline: 4.3381x; 1.0543x over previous
import functools

import jax
import jax.numpy as jnp
from jax import lax
from jax.experimental import pallas as pl
from jax.experimental.pallas import tpu as pltpu
from jax.experimental.pallas import tpu_sc as plsc

B = 16384
D = 32
T = 4
NE = 100000
NC = 2
NS = 16
NW = NC * NS
BPW = B // NW
CHUNK = 128
NIDX = BPW // CHUNK
NMETA = BPW * T
NMCH = NMETA // CHUNK

_mesh = plsc.VectorSubcoreMesh(core_axis_name="c", subcore_axis_name="s")


@functools.partial(
    pl.kernel,
    mesh=_mesh,
    out_type=jax.ShapeDtypeStruct((B, D), jnp.float32),
    compiler_params=pltpu.CompilerParams(use_tc_tiling_on_sc=False),
    scratch_types=[
        pltpu.VMEM((NIDX, CHUNK), jnp.int32),
        pltpu.VMEM((NMCH, CHUNK), jnp.int32),
        pltpu.VMEM((NMCH, CHUNK), jnp.int32),
        pltpu.VMEM((NMCH, CHUNK), jnp.float32),
        pltpu.VMEM((NMETA, D), jnp.float32),
        pltpu.VMEM((BPW, D), jnp.float32),
        pltpu.SemaphoreType.DMA,
        pltpu.SemaphoreType.DMA,
        pltpu.SemaphoreType.DMA,
    ],
)
def _sc_meta_sum(e2_hbm, tm_hbm, mif_hbm, mwf_hbm, ws_hbm,
                 idx_v, fidx_v, mi_v, mw_v, meta_v, ws_v,
                 sem_i, sem_w, sem_m):
    wid = lax.axis_index("s") * NC + lax.axis_index("c")
    base = wid * BPW

    pltpu.sync_copy(e2_hbm.at[pl.ds(wid * NIDX, NIDX)], idx_v)

    def flat_body(k, _):
        v = idx_v[k // 8, pl.ds((k % 8) * 16, 16)]
        for t in range(T):
            fidx_v[t * NIDX + k // 8, pl.ds((k % 8) * 16, 16)] = v + t * NE
        return 0

    lax.fori_loop(0, BPW // 16, flat_body, 0)

    mi_d, mw_d = [], []
    for j in range(NMCH):
        mi_d.append(pltpu.async_copy(
            mif_hbm.at[fidx_v.at[j]], mi_v.at[j], sem_i))
    for j in range(NMCH):
        mw_d.append(pltpu.async_copy(
            mwf_hbm.at[fidx_v.at[j]], mw_v.at[j], sem_w))
    for d in mi_d:
        d.wait()

    mt_d = []
    for j in range(NMCH):
        mt_d.append(pltpu.async_copy(
            tm_hbm.at[mi_v.at[j]], meta_v.at[pl.ds(j * CHUNK, CHUNK)], sem_m))
    for d in mw_d:
        d.wait()
    for d in mt_d:
        d.wait()

    def group_body(g, _):
        r = g // 8
        cb = (g % 8) * 16
        wv = [mw_v[t * NIDX + r, pl.ds(cb, 16)] for t in range(T)]
        for l in range(16):
            b = g * 16 + l
            w = [jnp.full((16,), wv[t][l]) for t in range(T)]
            for h in range(D // 16):
                sl = pl.ds(h * 16, 16)
                acc = w[0] * meta_v[b, sl]
                for t in range(1, T):
                    acc = acc + w[t] * meta_v[t * BPW + b, sl]
                ws_v[b, sl] = acc
        return 0

    lax.fori_loop(0, BPW // 16, group_body, 0)

    pltpu.sync_copy(ws_v, ws_hbm.at[pl.ds(base, BPW)])


@functools.partial(
    pl.kernel,
    mesh=_mesh,
    out_type=jax.ShapeDtypeStruct((B, D), jnp.float32),
    compiler_params=pltpu.CompilerParams(use_tc_tiling_on_sc=False),
    scratch_types=[
        pltpu.VMEM((NIDX, CHUNK), jnp.int32),
        pltpu.VMEM((BPW, D), jnp.float32),
        pltpu.VMEM((BPW, D), jnp.float32),
        pltpu.SemaphoreType.DMA,
        pltpu.SemaphoreType.DMA,
    ],
)
def _sc_elem_add(e2_hbm, te_hbm, ws_hbm, out_hbm,
                 idx_v, elem_v, ws_v, sem_e, sem_s):
    wid = lax.axis_index("s") * NC + lax.axis_index("c")
    base = wid * BPW

    pltpu.sync_copy(e2_hbm.at[pl.ds(wid * NIDX, NIDX)], idx_v)
    ws_d = pltpu.async_copy(ws_hbm.at[pl.ds(base, BPW)], ws_v, sem_s)
    el_d = []
    for j in range(NIDX):
        el_d.append(pltpu.async_copy(
            te_hbm.at[idx_v.at[j]], elem_v.at[pl.ds(j * CHUNK, CHUNK)], sem_e))
    for d in el_d:
        d.wait()
    ws_d.wait()

    scale = jnp.float32(0.2)

    def row_body(b, _):
        for h in range(D // 16):
            sl = pl.ds(h * 16, 16)
            elem_v[b, sl] = (elem_v[b, sl] + ws_v[b, sl]) * scale
        return 0

    lax.fori_loop(0, BPW, row_body, 0)

    pltpu.sync_copy(elem_v, out_hbm.at[pl.ds(base, BPW)])


def kernel(element_indicies, table_element, table_meta, meta_indicies,
           meta_weights):
    e2 = element_indicies.reshape(B // CHUNK, CHUNK)
    wsum = _sc_meta_sum(e2, table_meta,
                        meta_indicies.T.reshape(-1),
                        meta_weights.T.reshape(-1))
    return _sc_elem_add(e2, table_element, wsum)
